# Initial kernel scaffold; baseline (speedup 1.0000x reference)
import functools

import jax
import jax.numpy as jnp
from jax import lax
from jax.experimental import pallas as pl
from jax.experimental.pallas import tpu as pltpu
from jax.experimental.pallas import tpu_sc as plsc

N = 10000
E = 320000
D_IN = 128
D_HID = 16
NC = 2
NS = 16
NW = NC * NS
CHUNK = 128
CH = -(-E // (NW * CHUNK))
EPW = CH * CHUNK
EP = NW * EPW
NP = NW * 320
RPW = NP // NS
TRASH = N

_mesh = plsc.VectorSubcoreMesh(
    core_axis_name="c", subcore_axis_name="s", num_cores=NC, num_subcores=NS)


@functools.partial(
    pl.kernel,
    out_type=jax.ShapeDtypeStruct((NC, NP), jnp.float32),
    mesh=_mesh,
    scratch_types=[
        pltpu.VMEM((CH, CHUNK), jnp.int32),
        pltpu.VMEM((CHUNK,), jnp.float32),
        pltpu.VMEM((RPW,), jnp.float32),
        pltpu.VMEM_SHARED((NP,), jnp.float32),
    ],
)
def _deg_kernel(dst_hbm, out_hbm, dst_v, ones_v, zrow_v, deg_sh):
    cid = lax.axis_index("c")
    sid = lax.axis_index("s")
    w = cid * NS + sid
    pltpu.sync_copy(dst_hbm.at[w], dst_v)
    one16 = jnp.ones((16,), jnp.float32)
    zero16 = jnp.zeros((16,), jnp.float32)
    for i in range(CHUNK // 16):
        ones_v[pl.ds(i * 16, 16)] = one16
    for i in range(RPW // 16):
        zrow_v[pl.ds(i * 16, 16)] = zero16
    pltpu.sync_copy(zrow_v, deg_sh.at[pl.ds(sid * RPW, RPW)])
    plsc.subcore_barrier()
    for j in range(CH):
        pltpu.sync_copy(ones_v, deg_sh.at[dst_v.at[j]], add=True)
    plsc.subcore_barrier()
    pltpu.sync_copy(deg_sh.at[pl.ds(sid * RPW, RPW)],
                    out_hbm.at[cid, pl.ds(sid * RPW, RPW)])


@functools.partial(
    pl.kernel,
    out_type=jax.ShapeDtypeStruct((NC, NP, D_HID), jnp.float32),
    mesh=_mesh,
    scratch_types=[
        pltpu.VMEM((CH, CHUNK), jnp.int32),
        pltpu.VMEM((CH, CHUNK), jnp.int32),
        pltpu.VMEM((2, CHUNK, D_HID), jnp.float32),
        pltpu.VMEM((CHUNK, D_HID), jnp.float32),
        pltpu.VMEM_SHARED((NP, D_HID), jnp.float32),
        pltpu.SemaphoreType.DMA,
        pltpu.SemaphoreType.DMA,
    ],
)
def _pass_kernel(ht_hbm, src_hbm, dst_hbm, out_hbm,
                 src_v, dst_v, rows_v, ztile_v, acc_sh, sem0, sem1):
    cid = lax.axis_index("c")
    sid = lax.axis_index("s")
    w = cid * NS + sid
    pltpu.sync_copy(src_hbm.at[w], src_v)
    pltpu.sync_copy(dst_hbm.at[w], dst_v)
    zero16 = jnp.zeros((16,), jnp.float32)
    for i in range(CHUNK):
        ztile_v[i, :] = zero16
    for t in range(RPW // CHUNK):
        pltpu.sync_copy(ztile_v, acc_sh.at[pl.ds(sid * RPW + t * CHUNK, CHUNK)])
    plsc.subcore_barrier()

    sems = [sem0, sem1]
    desc = pltpu.async_copy(ht_hbm.at[src_v.at[0]], rows_v.at[0], sems[0])
    for j in range(CH):
        p = j % 2
        desc.wait()
        if j + 1 < CH:
            desc = pltpu.async_copy(
                ht_hbm.at[src_v.at[j + 1]], rows_v.at[1 - p], sems[1 - p])
        pltpu.sync_copy(rows_v.at[p], acc_sh.at[dst_v.at[j]], add=True)
    plsc.subcore_barrier()
    pltpu.sync_copy(acc_sh.at[pl.ds(sid * RPW, RPW)],
                    out_hbm.at[cid, pl.ds(sid * RPW, RPW)])


def _b_body(x_ref, w1_ref, degp_ref, ht_ref, dis_ref):
    deg = degp_ref[0, :] + degp_ref[1, :] + 1.0
    dis = lax.rsqrt(deg)
    dis_ref[...] = dis
    h = jnp.dot(x_ref[...], w1_ref[...], preferred_element_type=jnp.float32)
    ht_ref[...] = h * dis[:, None]


def _d_body(accp_ref, ht_ref, dis_ref, b1_ref, ht2_ref):
    acc = accp_ref[0] + accp_ref[1]
    dis = dis_ref[...][:, None]
    hr = jnp.maximum(dis * (acc + ht_ref[...]) + b1_ref[...][None, :], 0.0)
    ht2_ref[...] = dis * hr


def _f_body(accp_ref, ht2_ref, dis_ref, w2_ref, b2_ref, out_ref):
    acc = accp_ref[0] + accp_ref[1]
    agg = dis_ref[...][:, None] * (acc + ht2_ref[...])
    out_ref[...] = (
        jnp.dot(agg[:N], w2_ref[...], preferred_element_type=jnp.float32)
        + b2_ref[...][None, :])


def kernel(x, edge_index, W1, b1, W2, b2):
    src = edge_index[0]
    dst = edge_index[1]
    pad = EP - E
    src_p = jnp.concatenate([src, jnp.zeros((pad,), jnp.int32)]).reshape(
        NW, CH, CHUNK)
    dst_p = jnp.concatenate([dst, jnp.full((pad,), TRASH, jnp.int32)]).reshape(
        NW, CH, CHUNK)
    x_p = jnp.concatenate([x, jnp.zeros((NP - N, D_IN), jnp.float32)])

    degp = _deg_kernel(dst_p)

    ht1, dis = pl.pallas_call(
        _b_body,
        out_shape=(jax.ShapeDtypeStruct((NP, D_HID), jnp.float32),
                   jax.ShapeDtypeStruct((NP,), jnp.float32)),
    )(x_p, W1, degp)

    accp1 = _pass_kernel(ht1, src_p, dst_p)

    ht2 = pl.pallas_call(
        _d_body,
        out_shape=jax.ShapeDtypeStruct((NP, D_HID), jnp.float32),
    )(accp1, ht1, dis, b1)

    accp2 = _pass_kernel(ht2, src_p, dst_p)

    out = pl.pallas_call(
        _f_body,
        out_shape=jax.ShapeDtypeStruct((N, D_IN), jnp.float32),
    )(accp2, ht2, dis, W2, b2)
    return out


# speedup vs baseline: 39.7882x; 39.7882x over previous
import functools

import jax
import jax.numpy as jnp
from jax import lax
from jax.experimental import pallas as pl
from jax.experimental.pallas import tpu as pltpu
from jax.experimental.pallas import tpu_sc as plsc

N = 10000
E = 320000
D_IN = 128
D_HID = 16
NC = 2
NS = 16
NW = NC * NS
CHUNK = 128
CH = -(-E // (NW * CHUNK))
EPW = CH * CHUNK
EP = NW * EPW
NP = NW * 320
RPW = NP // NS
TRASH = N

_mesh = plsc.VectorSubcoreMesh(
    core_axis_name="c", subcore_axis_name="s", num_cores=NC, num_subcores=NS)


@functools.partial(
    pl.kernel,
    out_type=jax.ShapeDtypeStruct((NC, NP), jnp.float32),
    mesh=_mesh,
    scratch_types=[
        pltpu.VMEM((CH, CHUNK), jnp.int32),
        pltpu.VMEM((CHUNK,), jnp.float32),
        pltpu.VMEM((RPW,), jnp.float32),
        pltpu.VMEM_SHARED((NP,), jnp.float32),
    ],
    compiler_params=pltpu.CompilerParams(use_tc_tiling_on_sc=False),
)
def _deg_kernel(dst_hbm, out_hbm, dst_v, ones_v, zrow_v, deg_sh):
    cid = lax.axis_index("c")
    sid = lax.axis_index("s")
    w = cid * NS + sid
    pltpu.sync_copy(dst_hbm.at[w], dst_v)
    one16 = jnp.ones((16,), jnp.float32)
    zero16 = jnp.zeros((16,), jnp.float32)
    for i in range(CHUNK // 16):
        ones_v[pl.ds(i * 16, 16)] = one16
    for i in range(RPW // 16):
        zrow_v[pl.ds(i * 16, 16)] = zero16
    pltpu.sync_copy(zrow_v, deg_sh.at[pl.ds(sid * RPW, RPW)])
    plsc.subcore_barrier()
    for j in range(CH):
        pltpu.sync_copy(ones_v, deg_sh.at[dst_v.at[j]], add=True)
    plsc.subcore_barrier()
    pltpu.sync_copy(deg_sh.at[pl.ds(sid * RPW, RPW)],
                    out_hbm.at[cid, pl.ds(sid * RPW, RPW)])


@functools.partial(
    pl.kernel,
    out_type=jax.ShapeDtypeStruct((NC, NP, D_HID), jnp.float32),
    mesh=_mesh,
    scratch_types=[
        pltpu.VMEM((CH, CHUNK), jnp.int32),
        pltpu.VMEM((CH, CHUNK), jnp.int32),
        pltpu.VMEM((2, CHUNK, D_HID), jnp.float32),
        pltpu.VMEM((CHUNK, D_HID), jnp.float32),
        pltpu.VMEM_SHARED((NP, D_HID), jnp.float32),
        pltpu.SemaphoreType.DMA,
        pltpu.SemaphoreType.DMA,
    ],
    compiler_params=pltpu.CompilerParams(use_tc_tiling_on_sc=False),
)
def _pass_kernel(ht_hbm, src_hbm, dst_hbm, out_hbm,
                 src_v, dst_v, rows_v, ztile_v, acc_sh, sem0, sem1):
    cid = lax.axis_index("c")
    sid = lax.axis_index("s")
    w = cid * NS + sid
    pltpu.sync_copy(src_hbm.at[w], src_v)
    pltpu.sync_copy(dst_hbm.at[w], dst_v)
    zero16 = jnp.zeros((16,), jnp.float32)
    for i in range(CHUNK):
        ztile_v[i, :] = zero16
    for t in range(RPW // CHUNK):
        pltpu.sync_copy(ztile_v, acc_sh.at[pl.ds(sid * RPW + t * CHUNK, CHUNK)])
    plsc.subcore_barrier()

    sems = [sem0, sem1]
    desc = pltpu.async_copy(ht_hbm.at[src_v.at[0]], rows_v.at[0], sems[0])
    for j in range(CH):
        p = j % 2
        desc.wait()
        if j + 1 < CH:
            desc = pltpu.async_copy(
                ht_hbm.at[src_v.at[j + 1]], rows_v.at[1 - p], sems[1 - p])
        pltpu.sync_copy(rows_v.at[p], acc_sh.at[dst_v.at[j]], add=True)
    plsc.subcore_barrier()
    pltpu.sync_copy(acc_sh.at[pl.ds(sid * RPW, RPW)],
                    out_hbm.at[cid, pl.ds(sid * RPW, RPW)])


def _b_body(x_ref, w1_ref, degp_ref, ht_ref, dis_ref):
    deg = degp_ref[0, :] + degp_ref[1, :] + 1.0
    dis = lax.rsqrt(deg)
    dis_ref[...] = dis
    h = jnp.dot(x_ref[...], w1_ref[...], preferred_element_type=jnp.float32)
    ht_ref[...] = h * dis[:, None]


def _d_body(accp_ref, ht_ref, dis_ref, b1_ref, ht2_ref):
    acc = accp_ref[0] + accp_ref[1]
    dis = dis_ref[...][:, None]
    hr = jnp.maximum(dis * (acc + ht_ref[...]) + b1_ref[...][None, :], 0.0)
    ht2_ref[...] = dis * hr


def _f_body(accp_ref, ht2_ref, dis_ref, w2_ref, b2_ref, out_ref):
    acc = accp_ref[0] + accp_ref[1]
    agg = dis_ref[...][:, None] * (acc + ht2_ref[...])
    out_ref[...] = (
        jnp.dot(agg[:N], w2_ref[...], preferred_element_type=jnp.float32)
        + b2_ref[...][None, :])


def kernel(x, edge_index, W1, b1, W2, b2):
    src = edge_index[0]
    dst = edge_index[1]
    pad = EP - E
    src_p = jnp.concatenate([src, jnp.zeros((pad,), jnp.int32)]).reshape(
        NW, CH, CHUNK)
    dst_p = jnp.concatenate([dst, jnp.full((pad,), TRASH, jnp.int32)]).reshape(
        NW, CH, CHUNK)
    x_p = jnp.concatenate([x, jnp.zeros((NP - N, D_IN), jnp.float32)])

    degp = _deg_kernel(dst_p)

    ht1, dis = pl.pallas_call(
        _b_body,
        out_shape=(jax.ShapeDtypeStruct((NP, D_HID), jnp.float32),
                   jax.ShapeDtypeStruct((NP,), jnp.float32)),
    )(x_p, W1, degp)

    accp1 = _pass_kernel(ht1, src_p, dst_p)

    ht2 = pl.pallas_call(
        _d_body,
        out_shape=jax.ShapeDtypeStruct((NP, D_HID), jnp.float32),
    )(accp1, ht1, dis, b1)

    accp2 = _pass_kernel(ht2, src_p, dst_p)

    out = pl.pallas_call(
        _f_body,
        out_shape=jax.ShapeDtypeStruct((N, D_IN), jnp.float32),
    )(accp2, ht2, dis, W2, b2)
    return out


# speedup vs baseline: 55.3031x; 1.3899x over previous
import functools

import jax
import jax.numpy as jnp
from jax import lax
from jax.experimental import pallas as pl
from jax.experimental.pallas import tpu as pltpu
from jax.experimental.pallas import tpu_sc as plsc

N = 10000
E = 320000
D_IN = 128
D_HID = 16
NC = 2
NS = 16
NW = NC * NS
CHUNK = 128
CH = -(-E // (NW * CHUNK))
EPW = CH * CHUNK
EP = NW * EPW
NP = NW * 320
RPW = NP // NS
TRASH = N

_mesh = plsc.VectorSubcoreMesh(
    core_axis_name="c", subcore_axis_name="s", num_cores=NC, num_subcores=NS)


@functools.partial(
    pl.kernel,
    out_type=jax.ShapeDtypeStruct((NC, NP), jnp.float32),
    mesh=_mesh,
    scratch_types=[
        pltpu.VMEM((CH, CHUNK), jnp.int32),
        pltpu.VMEM((CHUNK,), jnp.float32),
        pltpu.VMEM((RPW,), jnp.float32),
        pltpu.VMEM_SHARED((NP,), jnp.float32),
    ],
    compiler_params=pltpu.CompilerParams(use_tc_tiling_on_sc=False),
)
def _deg_kernel(dst_hbm, out_hbm, dst_v, ones_v, zrow_v, deg_sh):
    cid = lax.axis_index("c")
    sid = lax.axis_index("s")
    w = cid * NS + sid
    pltpu.sync_copy(dst_hbm.at[w], dst_v)
    one16 = jnp.ones((16,), jnp.float32)
    zero16 = jnp.zeros((16,), jnp.float32)
    for i in range(CHUNK // 16):
        ones_v[pl.ds(i * 16, 16)] = one16
    for i in range(RPW // 16):
        zrow_v[pl.ds(i * 16, 16)] = zero16
    pltpu.sync_copy(zrow_v, deg_sh.at[pl.ds(sid * RPW, RPW)])
    plsc.subcore_barrier()
    for j in range(CH):
        pltpu.sync_copy(ones_v, deg_sh.at[dst_v.at[j]], add=True)
    plsc.subcore_barrier()
    pltpu.sync_copy(deg_sh.at[pl.ds(sid * RPW, RPW)],
                    out_hbm.at[cid, pl.ds(sid * RPW, RPW)])


@functools.partial(
    pl.kernel,
    out_type=jax.ShapeDtypeStruct((NC, NP, D_HID), jnp.float32),
    mesh=_mesh,
    scratch_types=[
        pltpu.VMEM((CH, CHUNK), jnp.int32),
        pltpu.VMEM((CH, CHUNK), jnp.int32),
        pltpu.VMEM((6, CHUNK, D_HID), jnp.float32),
        pltpu.VMEM((CHUNK, D_HID), jnp.float32),
        pltpu.VMEM_SHARED((NP, D_HID), jnp.float32),
        [pltpu.SemaphoreType.DMA] * 6,
        [pltpu.SemaphoreType.DMA] * 6,
    ],
    compiler_params=pltpu.CompilerParams(use_tc_tiling_on_sc=False),
)
def _pass_kernel(ht_hbm, src_hbm, dst_hbm, out_hbm,
                 src_v, dst_v, rows_v, ztile_v, acc_sh, gsems, ssems):
    cid = lax.axis_index("c")
    sid = lax.axis_index("s")
    w = cid * NS + sid
    pltpu.sync_copy(src_hbm.at[w], src_v)
    pltpu.sync_copy(dst_hbm.at[w], dst_v)
    zero16 = jnp.zeros((16,), jnp.float32)
    for i in range(CHUNK):
        ztile_v[i, :] = zero16
    for t in range(RPW // CHUNK):
        pltpu.sync_copy(ztile_v, acc_sh.at[pl.ds(sid * RPW + t * CHUNK, CHUNK)])
    plsc.subcore_barrier()

    NBUF, LAG = 6, 3
    gd = [None] * CH
    sd = [None] * CH
    for j in range(CH):
        b = j % NBUF
        if j >= NBUF:
            sd[j - NBUF].wait()
        gd[j] = pltpu.async_copy(ht_hbm.at[src_v.at[j]], rows_v.at[b],
                                 gsems[b])
        if j >= LAG:
            k = j - LAG
            gd[k].wait()
            sd[k] = pltpu.async_copy(rows_v.at[k % NBUF],
                                     acc_sh.at[dst_v.at[k]],
                                     ssems[k % NBUF], add=True)
    for k in range(CH - LAG, CH):
        gd[k].wait()
        sd[k] = pltpu.async_copy(rows_v.at[k % NBUF],
                                 acc_sh.at[dst_v.at[k]],
                                 ssems[k % NBUF], add=True)
    for k in range(CH - NBUF, CH):
        sd[k].wait()
    plsc.subcore_barrier()
    pltpu.sync_copy(acc_sh.at[pl.ds(sid * RPW, RPW)],
                    out_hbm.at[cid, pl.ds(sid * RPW, RPW)])


def _b_body(x_ref, w1_ref, degp_ref, ht_ref, dis_ref):
    deg = degp_ref[0, :] + degp_ref[1, :] + 1.0
    dis = lax.rsqrt(deg)
    dis_ref[...] = dis
    h = jnp.dot(x_ref[...], w1_ref[...], preferred_element_type=jnp.float32)
    ht_ref[...] = h * dis[:, None]


def _d_body(accp_ref, ht_ref, dis_ref, b1_ref, ht2_ref):
    acc = accp_ref[0] + accp_ref[1]
    dis = dis_ref[...][:, None]
    hr = jnp.maximum(dis * (acc + ht_ref[...]) + b1_ref[...][None, :], 0.0)
    ht2_ref[...] = dis * hr


def _f_body(accp_ref, ht2_ref, dis_ref, w2_ref, b2_ref, out_ref):
    acc = accp_ref[0] + accp_ref[1]
    agg = dis_ref[...][:, None] * (acc + ht2_ref[...])
    out_ref[...] = (
        jnp.dot(agg[:N], w2_ref[...], preferred_element_type=jnp.float32)
        + b2_ref[...][None, :])


def kernel(x, edge_index, W1, b1, W2, b2):
    src = edge_index[0]
    dst = edge_index[1]
    pad = EP - E
    src_p = jnp.concatenate([src, jnp.zeros((pad,), jnp.int32)]).reshape(
        NW, CH, CHUNK)
    dst_p = jnp.concatenate([dst, jnp.full((pad,), TRASH, jnp.int32)]).reshape(
        NW, CH, CHUNK)
    x_p = jnp.concatenate([x, jnp.zeros((NP - N, D_IN), jnp.float32)])

    degp = _deg_kernel(dst_p)

    ht1, dis = pl.pallas_call(
        _b_body,
        out_shape=(jax.ShapeDtypeStruct((NP, D_HID), jnp.float32),
                   jax.ShapeDtypeStruct((NP,), jnp.float32)),
    )(x_p, W1, degp)

    accp1 = _pass_kernel(ht1, src_p, dst_p)

    ht2 = pl.pallas_call(
        _d_body,
        out_shape=jax.ShapeDtypeStruct((NP, D_HID), jnp.float32),
    )(accp1, ht1, dis, b1)

    accp2 = _pass_kernel(ht2, src_p, dst_p)

    out = pl.pallas_call(
        _f_body,
        out_shape=jax.ShapeDtypeStruct((N, D_IN), jnp.float32),
    )(accp2, ht2, dis, W2, b2)
    return out


# speedup vs baseline: 57.8362x; 1.0458x over previous
import functools

import jax
import jax.numpy as jnp
from jax import lax
from jax.experimental import pallas as pl
from jax.experimental.pallas import tpu as pltpu
from jax.experimental.pallas import tpu_sc as plsc

N = 10000
E = 320000
D_IN = 128
D_HID = 16
NC = 2
NS = 16
NW = NC * NS
CHUNK = 128
CH = -(-E // (NW * CHUNK))
EPW = CH * CHUNK
EP = NW * EPW
NP = NW * 320
RPW = NP // NS
TRASH = N

_mesh = plsc.VectorSubcoreMesh(
    core_axis_name="c", subcore_axis_name="s", num_cores=NC, num_subcores=NS)


@functools.partial(
    pl.kernel,
    out_type=jax.ShapeDtypeStruct((NC, NP), jnp.float32),
    mesh=_mesh,
    scratch_types=[
        pltpu.VMEM((CH, CHUNK), jnp.int32),
        pltpu.VMEM((CHUNK,), jnp.float32),
        pltpu.VMEM((RPW,), jnp.float32),
        pltpu.VMEM_SHARED((NP,), jnp.float32),
        [pltpu.SemaphoreType.DMA] * 8,
    ],
    compiler_params=pltpu.CompilerParams(use_tc_tiling_on_sc=False),
)
def _deg_kernel(dst_hbm, out_hbm, dst_v, ones_v, zrow_v, deg_sh, sems):
    cid = lax.axis_index("c")
    sid = lax.axis_index("s")
    w = cid * NS + sid
    pltpu.sync_copy(dst_hbm.at[w], dst_v)
    one16 = jnp.ones((16,), jnp.float32)
    zero16 = jnp.zeros((16,), jnp.float32)
    for i in range(CHUNK // 16):
        ones_v[pl.ds(i * 16, 16)] = one16
    for i in range(RPW // 16):
        zrow_v[pl.ds(i * 16, 16)] = zero16
    pltpu.sync_copy(zrow_v, deg_sh.at[pl.ds(sid * RPW, RPW)])
    plsc.subcore_barrier()
    dd = [None] * CH
    for j in range(CH):
        if j >= 8:
            dd[j - 8].wait()
        dd[j] = pltpu.async_copy(ones_v, deg_sh.at[dst_v.at[j]],
                                 sems[j % 8], add=True)
    for j in range(CH - 8, CH):
        dd[j].wait()
    plsc.subcore_barrier()
    pltpu.sync_copy(deg_sh.at[pl.ds(sid * RPW, RPW)],
                    out_hbm.at[cid, pl.ds(sid * RPW, RPW)])


@functools.partial(
    pl.kernel,
    out_type=jax.ShapeDtypeStruct((NC, NP, D_HID), jnp.float32),
    mesh=_mesh,
    scratch_types=[
        pltpu.VMEM((CH, CHUNK), jnp.int32),
        pltpu.VMEM((CH, CHUNK), jnp.int32),
        pltpu.VMEM((10, CHUNK, D_HID), jnp.float32),
        pltpu.VMEM((CHUNK, D_HID), jnp.float32),
        pltpu.VMEM_SHARED((NP, D_HID), jnp.float32),
        [pltpu.SemaphoreType.DMA] * 10,
        [pltpu.SemaphoreType.DMA] * 10,
    ],
    compiler_params=pltpu.CompilerParams(use_tc_tiling_on_sc=False),
)
def _pass_kernel(ht_hbm, src_hbm, dst_hbm, out_hbm,
                 src_v, dst_v, rows_v, ztile_v, acc_sh, gsems, ssems):
    cid = lax.axis_index("c")
    sid = lax.axis_index("s")
    w = cid * NS + sid
    pltpu.sync_copy(src_hbm.at[w], src_v)
    pltpu.sync_copy(dst_hbm.at[w], dst_v)
    zero16 = jnp.zeros((16,), jnp.float32)
    for i in range(CHUNK):
        ztile_v[i, :] = zero16
    for t in range(RPW // CHUNK):
        pltpu.sync_copy(ztile_v, acc_sh.at[pl.ds(sid * RPW + t * CHUNK, CHUNK)])
    plsc.subcore_barrier()

    NBUF, LAG = 10, 5
    gd = [None] * CH
    sd = [None] * CH
    for j in range(CH):
        b = j % NBUF
        if j >= NBUF:
            sd[j - NBUF].wait()
        gd[j] = pltpu.async_copy(ht_hbm.at[src_v.at[j]], rows_v.at[b],
                                 gsems[b])
        if j >= LAG:
            k = j - LAG
            gd[k].wait()
            sd[k] = pltpu.async_copy(rows_v.at[k % NBUF],
                                     acc_sh.at[dst_v.at[k]],
                                     ssems[k % NBUF], add=True)
    for k in range(CH - LAG, CH):
        gd[k].wait()
        sd[k] = pltpu.async_copy(rows_v.at[k % NBUF],
                                 acc_sh.at[dst_v.at[k]],
                                 ssems[k % NBUF], add=True)
    for k in range(CH - NBUF, CH):
        sd[k].wait()
    plsc.subcore_barrier()
    pltpu.sync_copy(acc_sh.at[pl.ds(sid * RPW, RPW)],
                    out_hbm.at[cid, pl.ds(sid * RPW, RPW)])


def _b_body(x_ref, w1_ref, degp_ref, ht_ref, dis_ref):
    deg = degp_ref[0, :] + degp_ref[1, :] + 1.0
    dis = lax.rsqrt(deg)
    dis_ref[...] = dis
    h = jnp.dot(x_ref[...], w1_ref[...], preferred_element_type=jnp.float32)
    ht_ref[...] = h * dis[:, None]


def _d_body(accp_ref, ht_ref, dis_ref, b1_ref, ht2_ref):
    acc = accp_ref[0] + accp_ref[1]
    dis = dis_ref[...][:, None]
    hr = jnp.maximum(dis * (acc + ht_ref[...]) + b1_ref[...][None, :], 0.0)
    ht2_ref[...] = dis * hr


def _f_body(accp_ref, ht2_ref, dis_ref, w2_ref, b2_ref, out_ref):
    acc = accp_ref[0] + accp_ref[1]
    agg = dis_ref[...][:, None] * (acc + ht2_ref[...])
    out_ref[...] = (
        jnp.dot(agg[:N], w2_ref[...], preferred_element_type=jnp.float32)
        + b2_ref[...][None, :])


def kernel(x, edge_index, W1, b1, W2, b2):
    src = edge_index[0]
    dst = edge_index[1]
    pad = EP - E
    src_p = jnp.concatenate([src, jnp.zeros((pad,), jnp.int32)]).reshape(
        NW, CH, CHUNK)
    dst_p = jnp.concatenate([dst, jnp.full((pad,), TRASH, jnp.int32)]).reshape(
        NW, CH, CHUNK)
    x_p = jnp.concatenate([x, jnp.zeros((NP - N, D_IN), jnp.float32)])

    degp = _deg_kernel(dst_p)

    ht1, dis = pl.pallas_call(
        _b_body,
        out_shape=(jax.ShapeDtypeStruct((NP, D_HID), jnp.float32),
                   jax.ShapeDtypeStruct((NP,), jnp.float32)),
    )(x_p, W1, degp)

    accp1 = _pass_kernel(ht1, src_p, dst_p)

    ht2 = pl.pallas_call(
        _d_body,
        out_shape=jax.ShapeDtypeStruct((NP, D_HID), jnp.float32),
    )(accp1, ht1, dis, b1)

    accp2 = _pass_kernel(ht2, src_p, dst_p)

    out = pl.pallas_call(
        _f_body,
        out_shape=jax.ShapeDtypeStruct((N, D_IN), jnp.float32),
    )(accp2, ht2, dis, W2, b2)
    return out
